# initial kernel scaffold (unmeasured)
import jax
import jax.numpy as jnp
from jax import lax
from jax.experimental import pallas as pl
from jax.experimental.pallas import tpu as pltpu

N_ROWS = 2048
N_COLS = 1024


def _exchange(x_send):

    def body(x_ref, out_ref, send_sem, recv_sem):
        my_x = lax.axis_index("x")
        my_y = lax.axis_index("y")
        my_z = lax.axis_index("z")
        peer = (1 - my_x, my_y, my_z)

        barrier_sem = pltpu.get_barrier_semaphore()
        pl.semaphore_signal(
            barrier_sem, inc=1, device_id=peer,
            device_id_type=pl.DeviceIdType.MESH,
        )
        pl.semaphore_wait(barrier_sem, 1)

        rdma = pltpu.make_async_remote_copy(
            src_ref=x_ref,
            dst_ref=out_ref,
            send_sem=send_sem,
            recv_sem=recv_sem,
            device_id=peer,
            device_id_type=pl.DeviceIdType.MESH,
        )
        rdma.start()
        rdma.wait()

    return pl.pallas_call(
        body,
        out_shape=jax.ShapeDtypeStruct((N_ROWS, N_COLS), jnp.float32),
        in_specs=[pl.BlockSpec(memory_space=pltpu.VMEM)],
        out_specs=pl.BlockSpec(memory_space=pltpu.VMEM),
        scratch_shapes=[
            pltpu.SemaphoreType.DMA,
            pltpu.SemaphoreType.DMA,
        ],
        compiler_params=pltpu.CompilerParams(collective_id=0),
    )(x_send)


def kernel(x, dest):
    p = lax.axis_index("x")
    n = N_ROWS

    keep_mask = dest == p
    n_keep = jnp.sum(keep_mask.astype(jnp.int32))
    n_ex = n - n_keep

    keep_first = jnp.argsort(jnp.logical_not(keep_mask), stable=True)
    send_first = jnp.argsort(keep_mask, stable=True)
    x_keep = x[keep_first]
    x_send = x[send_first]

    recv = _exchange(x_send)

    r = jnp.arange(n)
    gi = jnp.where(
        p == 0,
        jnp.where(r < n_keep, r, n + r - n_keep),
        jnp.where(r < n_ex, n + r, r - n_ex),
    )
    combined = jnp.concatenate([x_keep, recv], axis=0)
    return combined[gi]


# baseline (device time: 108136 ns/iter reference)
import jax
import jax.numpy as jnp
from jax import lax
from jax.experimental import pallas as pl
from jax.experimental.pallas import tpu as pltpu

N_ROWS = 2048
N_COLS = 1024
BUF_ROWS = N_ROWS + 8

_CHUNKS = [128] * 16 + [8] * 16
N_CHUNKS = len(_CHUNKS)


def _chunk_plan(n_ex):
    a = n_ex // 128
    r = n_ex - a * 128
    b = (r + 7) // 8
    plan = []
    for k in range(16):
        plan.append((k < a, k * 128))
    for j in range(16):
        plan.append((j < b, a * 128 + j * 8))
    return plan


def _a2av(x, dest, off, meta):
    def body(x_ref, dest_ref, off_ref, meta_ref, out_ref, send_ref,
             recv_ref, send_sems, recv_sems):
        my_x = lax.axis_index("x")
        my_y = lax.axis_index("y")
        my_z = lax.axis_index("z")
        peer = (1 - my_x, my_y, my_z)

        n_ex = meta_ref[0]
        my_base = meta_ref[1]

        barrier_sem = pltpu.get_barrier_semaphore()
        pl.semaphore_signal(
            barrier_sem, inc=1, device_id=peer,
            device_id_type=pl.DeviceIdType.MESH,
        )
        pl.semaphore_wait(barrier_sem, 1)

        def row(i, carry):
            d = dest_ref[i]
            o = off_ref[i]

            @pl.when(d == my_x)
            def _():
                out_ref[pl.ds(o, 1), :] = x_ref[pl.ds(i, 1), :]

            @pl.when(d != my_x)
            def _():
                send_ref[pl.ds(o, 1), :] = x_ref[pl.ds(i, 1), :]

            return carry

        lax.fori_loop(0, N_ROWS, row, 0)

        plan = _chunk_plan(n_ex)

        for idx, (size, (cond, start)) in enumerate(zip(_CHUNKS, plan)):
            @pl.when(cond)
            def _(idx=idx, size=size, start=start):
                rdma = pltpu.make_async_remote_copy(
                    src_ref=send_ref.at[pl.ds(start, size)],
                    dst_ref=recv_ref.at[pl.ds(start, size)],
                    send_sem=send_sems.at[idx],
                    recv_sem=recv_sems.at[idx],
                    device_id=peer,
                    device_id_type=pl.DeviceIdType.MESH,
                )
                rdma.start()

        for idx, (size, (cond, start)) in enumerate(zip(_CHUNKS, plan)):
            @pl.when(cond)
            def _(idx=idx, size=size, start=start):
                desc = pltpu.make_async_remote_copy(
                    src_ref=send_ref.at[pl.ds(start, size)],
                    dst_ref=recv_ref.at[pl.ds(start, size)],
                    send_sem=send_sems.at[idx],
                    recv_sem=recv_sems.at[idx],
                    device_id=peer,
                    device_id_type=pl.DeviceIdType.MESH,
                )
                desc.wait_send()
                desc.wait_recv()

        def unpack(i, carry):
            out_ref[pl.ds(my_base + i, 1), :] = recv_ref[pl.ds(i, 1), :]
            return carry

        lax.fori_loop(0, n_ex, unpack, 0)

    return pl.pallas_call(
        body,
        out_shape=jax.ShapeDtypeStruct((N_ROWS, N_COLS), jnp.float32),
        in_specs=[
            pl.BlockSpec(memory_space=pltpu.VMEM),
            pl.BlockSpec(memory_space=pltpu.SMEM),
            pl.BlockSpec(memory_space=pltpu.SMEM),
            pl.BlockSpec(memory_space=pltpu.SMEM),
        ],
        out_specs=pl.BlockSpec(memory_space=pltpu.VMEM),
        scratch_shapes=[
            pltpu.VMEM((BUF_ROWS, N_COLS), jnp.float32),
            pltpu.VMEM((BUF_ROWS, N_COLS), jnp.float32),
            pltpu.SemaphoreType.DMA((N_CHUNKS,)),
            pltpu.SemaphoreType.DMA((N_CHUNKS,)),
        ],
        compiler_params=pltpu.CompilerParams(collective_id=0),
    )(x, dest, off, meta)


def kernel(x, dest):
    p = lax.axis_index("x")
    keep = (dest == p).astype(jnp.int32)
    n_keep = jnp.sum(keep)
    n_ex = N_ROWS - n_keep

    kpre = jnp.cumsum(keep) - keep
    spre = jnp.cumsum(1 - keep) - (1 - keep)

    base_keep = jnp.where(p == 0, 0, n_ex)
    my_base = jnp.where(p == 0, n_keep, 0)

    off = jnp.where(keep == 1, base_keep + kpre, spre).astype(jnp.int32)
    meta = jnp.stack([n_ex, my_base]).astype(jnp.int32)

    return _a2av(x, dest.astype(jnp.int32), off, meta)


# device time: 79122 ns/iter; 1.3667x vs baseline; 1.3667x over previous
import jax
import jax.numpy as jnp
from jax import lax
from jax.experimental import pallas as pl
from jax.experimental.pallas import tpu as pltpu

N_ROWS = 2048
N_COLS = 1024
BUF_ROWS = N_ROWS + 8

_CHUNKS = [128] * 16 + [8] * 16
N_CHUNKS = len(_CHUNKS)


def _chunk_plan(n_ex):
    a = n_ex // 128
    r = n_ex - a * 128
    b = (r + 7) // 8
    plan = []
    for k in range(16):
        plan.append((k < a, k * 128))
    for j in range(16):
        plan.append((j < b, a * 128 + j * 8))
    return plan


def _a2av(x, dest, off, meta):
    def body(x_ref, dest_ref, off_ref, meta_ref, out_ref, send_ref,
             recv_ref, send_sems, recv_sems):
        my_x = lax.axis_index("x")
        my_y = lax.axis_index("y")
        my_z = lax.axis_index("z")
        peer = (1 - my_x, my_y, my_z)

        n_ex = meta_ref[0]
        my_base = meta_ref[1]

        barrier_sem = pltpu.get_barrier_semaphore()
        pl.semaphore_signal(
            barrier_sem, inc=1, device_id=peer,
            device_id_type=pl.DeviceIdType.MESH,
        )
        pl.semaphore_wait(barrier_sem, 1)

        plan = _chunk_plan(n_ex)

        def make_desc(idx, size, start):
            return pltpu.make_async_remote_copy(
                src_ref=send_ref.at[pl.ds(start, size)],
                dst_ref=recv_ref.at[pl.ds(start, size)],
                send_sem=send_sems.at[idx],
                recv_sem=recv_sems.at[idx],
                device_id=peer,
                device_id_type=pl.DeviceIdType.MESH,
            )

        def row(i, carry):
            d = dest_ref[i]
            o = off_ref[i]

            @pl.when(d == my_x)
            def _():
                out_ref[pl.ds(o, 1), :] = x_ref[pl.ds(i, 1), :]

            @pl.when(d != my_x)
            def _():
                send_ref[pl.ds(o, 1), :] = x_ref[pl.ds(i, 1), :]

            return carry

        prev = 0
        for k in range(16):
            end = meta_ref[2 + k]
            lax.fori_loop(prev, end, row, 0)
            cond, start = plan[k]

            @pl.when(cond)
            def _(idx=k, start=start):
                make_desc(idx, 128, start).start()

            prev = end
        lax.fori_loop(prev, N_ROWS, row, 0)
        for j in range(16):
            cond, start = plan[16 + j]

            @pl.when(cond)
            def _(idx=16 + j, start=start):
                make_desc(idx, 8, start).start()

        def unpack(i, carry):
            out_ref[pl.ds(my_base + i, 1), :] = recv_ref[pl.ds(i, 1), :]
            return carry

        for idx, (size, (cond, start)) in enumerate(zip(_CHUNKS, plan)):
            @pl.when(cond)
            def _(idx=idx, size=size, start=start):
                make_desc(idx, size, start).wait_recv()

        lax.fori_loop(0, n_ex, unpack, 0)

        for idx, (size, (cond, start)) in enumerate(zip(_CHUNKS, plan)):
            @pl.when(cond)
            def _(idx=idx, size=size, start=start):
                make_desc(idx, size, start).wait_send()

    return pl.pallas_call(
        body,
        out_shape=jax.ShapeDtypeStruct((N_ROWS, N_COLS), jnp.float32),
        in_specs=[
            pl.BlockSpec(memory_space=pltpu.VMEM),
            pl.BlockSpec(memory_space=pltpu.SMEM),
            pl.BlockSpec(memory_space=pltpu.SMEM),
            pl.BlockSpec(memory_space=pltpu.SMEM),
        ],
        out_specs=pl.BlockSpec(memory_space=pltpu.VMEM),
        scratch_shapes=[
            pltpu.VMEM((BUF_ROWS, N_COLS), jnp.float32),
            pltpu.VMEM((BUF_ROWS, N_COLS), jnp.float32),
            pltpu.SemaphoreType.DMA((N_CHUNKS,)),
            pltpu.SemaphoreType.DMA((N_CHUNKS,)),
        ],
        compiler_params=pltpu.CompilerParams(collective_id=0),
    )(x, dest, off, meta)


def kernel(x, dest):
    p = lax.axis_index("x")
    keep = (dest == p).astype(jnp.int32)
    n_keep = jnp.sum(keep)
    n_ex = N_ROWS - n_keep

    kpre = jnp.cumsum(keep) - keep
    spre = jnp.cumsum(1 - keep) - (1 - keep)

    base_keep = jnp.where(p == 0, 0, n_ex)
    my_base = jnp.where(p == 0, n_keep, 0)

    off = jnp.where(keep == 1, base_keep + kpre, spre).astype(jnp.int32)

    scount = jnp.cumsum(1 - keep)
    targets = 128 * (jnp.arange(16, dtype=jnp.int32) + 1)
    trig = jnp.minimum(
        jnp.searchsorted(scount, targets, side="left") + 1, N_ROWS
    )
    meta = jnp.concatenate(
        [jnp.stack([n_ex, my_base]), trig]
    ).astype(jnp.int32)

    return _a2av(x, dest.astype(jnp.int32), off, meta)


# device time: 70766 ns/iter; 1.5281x vs baseline; 1.1181x over previous
import jax
import jax.numpy as jnp
from jax import lax
from jax.experimental import pallas as pl
from jax.experimental.pallas import tpu as pltpu

N_ROWS = 2048
N_COLS = 1024
BUF_ROWS = N_ROWS + 16

_CHUNKS = [128] * 16 + [8] * 17
N_CHUNKS = len(_CHUNKS)


def _chunk_plan(total):
    a = total // 128
    r = total - a * 128
    b = (r + 7) // 8
    plan = []
    for k in range(16):
        plan.append((k < a, k * 128))
    for j in range(17):
        plan.append((j < b, a * 128 + j * 8))
    return plan


def _a2av(x, keep_src, send_src, meta):
    def body(x_ref, keep_src_ref, send_src_ref, meta_ref, out_ref,
             send_ref, recv_ref, send_sems, recv_sems):
        my_x = lax.axis_index("x")
        my_y = lax.axis_index("y")
        my_z = lax.axis_index("z")
        peer = (1 - my_x, my_y, my_z)

        n_keep = meta_ref[0]
        n_ex = meta_ref[1]
        base_keep = meta_ref[2]
        my_base = meta_ref[3]
        phi = meta_ref[4]
        delta = meta_ref[5]

        barrier_sem = pltpu.get_barrier_semaphore()
        pl.semaphore_signal(
            barrier_sem, inc=1, device_id=peer,
            device_id_type=pl.DeviceIdType.MESH,
        )
        pl.semaphore_wait(barrier_sem, 1)

        plan_s = _chunk_plan(phi + n_ex)
        plan_r = _chunk_plan(delta + n_ex)

        def make_desc(idx, size, start):
            return pltpu.make_async_remote_copy(
                src_ref=send_ref.at[pl.ds(start, size)],
                dst_ref=recv_ref.at[pl.ds(start, size)],
                send_sem=send_sems.at[idx],
                recv_sem=recv_sems.at[idx],
                device_id=peer,
                device_id_type=pl.DeviceIdType.MESH,
            )

        def pack(t, carry):
            send_ref[pl.ds(phi + t, 1), :] = (
                x_ref[pl.ds(send_src_ref[t], 1), :]
            )
            return carry

        prev = jnp.int32(0)
        for k in range(16):
            seg_end = jnp.clip((k + 1) * 128 - phi, 0, n_ex)
            lax.fori_loop(prev, seg_end, pack, 0)
            cond, start = plan_s[k]

            @pl.when(cond)
            def _(idx=k, start=start):
                make_desc(idx, 128, start).start()

            prev = seg_end
        lax.fori_loop(prev, n_ex, pack, 0)
        for j in range(17):
            cond, start = plan_s[16 + j]

            @pl.when(cond)
            def _(idx=16 + j, start=start):
                make_desc(idx, 8, start).start()

        def keep_row(t, carry):
            out_ref[pl.ds(base_keep + t, 1), :] = (
                x_ref[pl.ds(keep_src_ref[t], 1), :]
            )
            return carry

        lax.fori_loop(0, n_keep, keep_row, 0)

        for idx, (size, (cond, start)) in enumerate(zip(_CHUNKS, plan_r)):
            @pl.when(cond)
            def _(idx=idx, size=size, start=start):
                make_desc(idx, size, start).wait_recv()

        base_al = pl.multiple_of(my_base - delta, 8)

        def unpack_row(u, carry):
            out_ref[pl.ds(my_base + (u - delta), 1), :] = (
                recv_ref[pl.ds(u, 1), :]
            )
            return carry

        def unpack_block(q, carry):
            out_ref[pl.ds(base_al + 8 * q, 8), :] = (
                recv_ref[pl.ds(8 * q, 8), :]
            )
            return carry

        total_r = delta + n_ex
        head_end = jnp.minimum(8, total_r)
        lax.fori_loop(delta, head_end, unpack_row, 0)
        big_q = total_r // 8
        lax.fori_loop(1, big_q, unpack_block, 0)
        tail_start = jnp.maximum(8 * big_q, head_end)
        lax.fori_loop(tail_start, total_r, unpack_row, 0)

        for idx, (size, (cond, start)) in enumerate(zip(_CHUNKS, plan_s)):
            @pl.when(cond)
            def _(idx=idx, size=size, start=start):
                make_desc(idx, size, start).wait_send()

    return pl.pallas_call(
        body,
        out_shape=jax.ShapeDtypeStruct((N_ROWS, N_COLS), jnp.float32),
        in_specs=[
            pl.BlockSpec(memory_space=pltpu.VMEM),
            pl.BlockSpec(memory_space=pltpu.SMEM),
            pl.BlockSpec(memory_space=pltpu.SMEM),
            pl.BlockSpec(memory_space=pltpu.SMEM),
        ],
        out_specs=pl.BlockSpec(memory_space=pltpu.VMEM),
        scratch_shapes=[
            pltpu.VMEM((BUF_ROWS, N_COLS), jnp.float32),
            pltpu.VMEM((BUF_ROWS, N_COLS), jnp.float32),
            pltpu.SemaphoreType.DMA((N_CHUNKS,)),
            pltpu.SemaphoreType.DMA((N_CHUNKS,)),
        ],
        compiler_params=pltpu.CompilerParams(collective_id=0),
    )(x, keep_src, send_src, meta)


def kernel(x, dest):
    p = lax.axis_index("x")
    keep = (dest == p).astype(jnp.int32)
    n_keep = jnp.sum(keep)
    n_ex = N_ROWS - n_keep

    keep_src = jnp.argsort(1 - keep, stable=True).astype(jnp.int32)
    send_src = jnp.argsort(keep, stable=True).astype(jnp.int32)

    base_keep = jnp.where(p == 0, 0, n_ex)
    my_base = jnp.where(p == 0, n_keep, 0)

    delta = my_base % 8
    phi = jnp.where(p == 0, 0, n_keep % 8)

    meta = jnp.stack(
        [n_keep, n_ex, base_keep, my_base, phi, delta]
    ).astype(jnp.int32)
    return _a2av(x, keep_src, send_src, meta)
